# baseline (speedup 1.0000x reference)
import functools

import jax
import jax.numpy as jnp
from jax import lax
from jax.experimental import pallas as pl
from jax.experimental.pallas import tpu as pltpu
from jax.experimental.pallas import tpu_sc as plsc

_N = 4096
_M = 8192
_D = 128
_C = 10
_RB = 2048
_CB = 512
_K = 16
_KV = 11
_NW = 32
_RPW = _M // _NW
_INF = 3.0e38
_BIGI = 1 << 30


def _topk_body(feats_ref, featsT_ref, pd_ref, idx_ref, b2_ref, fth_ref, ftl_ref):
    A = feats_ref[...]
    a2p = jnp.sum(A * A, axis=1, keepdims=True) + 1.0
    ah = A.astype(jnp.bfloat16)
    al = (A - ah.astype(jnp.float32)).astype(jnp.bfloat16)

    @pl.when(pl.program_id(0) == 0)
    def _():
        FT = featsT_ref[...]
        b2_ref[...] = jnp.sum(FT * FT, axis=0, keepdims=True)
        fh = FT.astype(jnp.bfloat16)
        fth_ref[...] = fh
        ftl_ref[...] = (FT - fh.astype(jnp.float32)).astype(jnp.bfloat16)

    lane_k = lax.broadcasted_iota(jnp.int32, (_RB, _K), 1)
    fpos = lax.broadcasted_iota(jnp.int32, (_RB, 128), 1) + _K
    init_k = lax.bitcast_convert_type(
        jnp.bitwise_or(jnp.full((_RB, _K), 0x7F000000, jnp.int32), lane_k),
        jnp.float32)
    nch = _M // _CB

    def _dot_chunk(c):
        bth = fth_ref[:, pl.ds(c * _CB, _CB)]
        btl = ftl_ref[:, pl.ds(c * _CB, _CB)]
        dims = (((1,), (0,)), ((), ()))
        return (lax.dot_general(ah, bth, dims, preferred_element_type=jnp.float32)
                + lax.dot_general(ah, btl, dims, preferred_element_type=jnp.float32)
                + lax.dot_general(al, bth, dims, preferred_element_type=jnp.float32))

    def chunk_body(c, carry):
        run_k, run_i, S = carry
        S_next = _dot_chunk(jnp.minimum(c + 1, nch - 1))
        b2 = b2_ref[:, pl.ds(c * _CB, _CB)]
        d2p = a2p + b2 - 2.0 * S
        s0, s1 = d2p[:, 0:128], d2p[:, 128:256]
        s2, s3 = d2p[:, 256:384], d2p[:, 384:512]
        v = jnp.minimum(jnp.minimum(s0, s1), jnp.minimum(s2, s3))
        colid = jnp.where(v == s0, 0,
                          jnp.where(v == s1, 1, jnp.where(v == s2, 2, 3)))
        qb = jnp.bitwise_and(
            lax.bitcast_convert_type(v, jnp.int32) + 512, ~1023)
        fm = lax.bitcast_convert_type(
            jnp.bitwise_or(qb, fpos + jnp.left_shift(colid, 7)), jnp.float32)
        rk = lax.bitcast_convert_type(jnp.bitwise_or(jnp.bitwise_and(
            lax.bitcast_convert_type(run_k, jnp.int32), ~1023), lane_k),
            jnp.float32)
        cand = jnp.concatenate([rk, fm], axis=1)
        new_k = init_k
        for t in range(_KV):
            mk = jnp.min(cand, axis=1, keepdims=True)
            new_k = jnp.where(lane_k == t, mk, new_k)
            if t + 1 < _KV:
                cand = jnp.where(cand == mk, _INF, cand)
        npos = jnp.bitwise_and(lax.bitcast_convert_type(new_k, jnp.int32), 1023)
        new_i = jnp.where(npos >= _K, c * _CB + (npos - _K), 0)
        for s in range(_KV):
            new_i = jnp.where(npos == s, run_i[:, s:s + 1], new_i)
        return new_k, new_i, S_next

    init = (init_k, jnp.zeros((_RB, _K), jnp.int32), _dot_chunk(0))
    run_k, run_i, _ = lax.fori_loop(0, nch, chunk_body, init)
    d2q = lax.bitcast_convert_type(
        jnp.bitwise_and(lax.bitcast_convert_type(run_k, jnp.int32), ~1023),
        jnp.float32) - 1.0
    pdv = jnp.sqrt(jnp.maximum(d2q, 0.0))
    pd_ref[...] = jnp.where(lane_k < _KV, pdv, 0.0)
    idx_ref[...] = run_i


_topk_call = pl.pallas_call(
    _topk_body,
    grid=(_M // _RB,),
    in_specs=[
        pl.BlockSpec((_RB, _D), lambda i: (i, 0)),
        pl.BlockSpec((_D, _M), lambda i: (0, 0)),
    ],
    out_specs=[
        pl.BlockSpec((_RB, _K), lambda i: (i, 0)),
        pl.BlockSpec((_RB, _K), lambda i: (i, 0)),
    ],
    out_shape=[
        jax.ShapeDtypeStruct((_M, _K), jnp.float32),
        jax.ShapeDtypeStruct((_M, _K), jnp.int32),
    ],
    scratch_shapes=[pltpu.VMEM((1, _M), jnp.float32),
                    pltpu.VMEM((_D, _M), jnp.bfloat16),
                    pltpu.VMEM((_D, _M), jnp.bfloat16)],
)


def _sc_body(pd_hbm, idx_hbm, probs_hbm, lab_hbm, out_hbm,
             pd_v, idx_v, probs_v, lab_v, acc_v):
    nc = plsc.get_sparse_core_info().num_cores
    wid = lax.axis_index("s") * nc + lax.axis_index("c")
    pltpu.sync_copy(pd_hbm.at[wid], pd_v)
    pltpu.sync_copy(idx_hbm.at[wid], idx_v)
    pltpu.sync_copy(probs_hbm, probs_v)
    pltpu.sync_copy(lab_hbm, lab_v)
    base = wid * _RPW
    lanes = lax.iota(jnp.int32, 16)

    def group_body(g, acc):
        off = g * 16
        i_vec = base + off + lanes
        imod = jnp.bitwise_and(i_vec, _N - 1)
        lab_i = plsc.load_gather(lab_v, [imod])
        ok_i = lab_i != -1
        r = pd_v[pl.ds(_RPW + off, 16)]
        contrib = jnp.zeros((16,), jnp.float32)
        for k in range(_KV):
            pdk = pd_v[pl.ds(k * _RPW + off, 16)]
            idxk = idx_v[pl.ds(k * _RPW + off, 16)]
            jmod = jnp.bitwise_and(idxk, _N - 1)
            lab_j = plsc.load_gather(lab_v, [jmod])
            msk = (lab_i == lab_j) & (imod != jmod) & ok_i & (lab_j != -1)
            w = 1.0 - jnp.clip((pdk - r) / r, 0.0, 1.0)
            w = jnp.where(idxk == i_vec, 0.0, w)
            w = jnp.where(msk, w, 0.0)
            dot = jnp.zeros((16,), jnp.float32)
            for ch in range(_C):
                pic = plsc.load_gather(probs_v, [i_vec + ch * _M])
                pjc = plsc.load_gather(probs_v, [idxk + ch * _M])
                dot = dot + pic * pjc
            contrib = contrib + w * dot
        return acc + contrib

    acc = lax.fori_loop(0, _RPW // 16, group_body, jnp.zeros((16,), jnp.float32))
    acc_v[...] = acc
    pltpu.sync_copy(acc_v, out_hbm.at[wid])


def _make_sc_call():
    return functools.partial(
        pl.kernel,
        out_type=jax.ShapeDtypeStruct((_NW, 16), jnp.float32),
        mesh=plsc.VectorSubcoreMesh(core_axis_name="c", subcore_axis_name="s"),
        compiler_params=pltpu.CompilerParams(needs_layout_passes=False),
        scratch_types=[
            pltpu.VMEM((_K * _RPW,), jnp.float32),
            pltpu.VMEM((_K * _RPW,), jnp.int32),
            pltpu.VMEM((_C * _M,), jnp.float32),
            pltpu.VMEM((_N,), jnp.int32),
            pltpu.VMEM((16,), jnp.float32),
        ],
    )(_sc_body)


def kernel(z_i, z_j, z_i_prob, z_j_prob, pseudo_label):
    feats = jnp.concatenate([z_i, z_j], axis=0)
    probs = jnp.concatenate([z_i_prob, z_j_prob], axis=0)
    pd, idx = _topk_call(feats, feats.T)
    pd_w = pd.T.reshape(_K, _NW, _RPW).transpose(1, 0, 2).reshape(_NW, _K * _RPW)
    idx_w = idx.T.reshape(_K, _NW, _RPW).transpose(1, 0, 2).reshape(_NW, _K * _RPW)
    probs_f = probs.T.reshape(-1)
    partials = _make_sc_call()(pd_w, idx_w, probs_f, pseudo_label)
    return jnp.sum(partials) / jnp.float32(_M * _M)


# speedup vs baseline: 1.5116x; 1.5116x over previous
import functools

import jax
import jax.numpy as jnp
from jax import lax
from jax.experimental import pallas as pl
from jax.experimental.pallas import tpu as pltpu
from jax.experimental.pallas import tpu_sc as plsc

_N = 4096
_M = 8192
_D = 128
_C = 10
_RB = 1024
_CB = 1024
_PM = 2047
_K = 16
_KV = 11
_NW = 32
_RPW = _M // _NW
_INF = 3.0e38
_BIGI = 1 << 30


def _topk_body(feats_ref, featsT_ref, pd_ref, idx_ref, b2_ref, fth_ref, ftl_ref):
    A = feats_ref[...]
    a2p = jnp.sum(A * A, axis=1, keepdims=True) + 1.0
    ah = A.astype(jnp.bfloat16)
    al = (A - ah.astype(jnp.float32)).astype(jnp.bfloat16)

    @pl.when(pl.program_id(0) == 0)
    def _():
        FT = featsT_ref[...]
        b2_ref[...] = jnp.sum(FT * FT, axis=0, keepdims=True)
        fh = FT.astype(jnp.bfloat16)
        fth_ref[...] = fh
        ftl_ref[...] = (FT - fh.astype(jnp.float32)).astype(jnp.bfloat16)

    lane_k = lax.broadcasted_iota(jnp.int32, (_RB, _K), 1)
    fpos = lax.broadcasted_iota(jnp.int32, (_RB, 128), 1) + _K
    init_k = lax.bitcast_convert_type(
        jnp.bitwise_or(jnp.full((_RB, _K), 0x7F000000, jnp.int32), lane_k),
        jnp.float32)
    nch = _M // _CB

    def _dot_chunk(c):
        bth = fth_ref[:, pl.ds(c * _CB, _CB)]
        btl = ftl_ref[:, pl.ds(c * _CB, _CB)]
        dims = (((1,), (0,)), ((), ()))
        return (lax.dot_general(ah, bth, dims, preferred_element_type=jnp.float32)
                + lax.dot_general(ah, btl, dims, preferred_element_type=jnp.float32)
                + lax.dot_general(al, bth, dims, preferred_element_type=jnp.float32))

    def chunk_body(c, carry):
        run_k, run_i, S = carry
        S_next = _dot_chunk(jnp.minimum(c + 1, nch - 1))
        b2 = b2_ref[:, pl.ds(c * _CB, _CB)]
        d2p = a2p + b2 - 2.0 * S
        sl = [d2p[:, j * 128:(j + 1) * 128] for j in range(8)]
        v = jnp.minimum(
            jnp.minimum(jnp.minimum(sl[0], sl[1]), jnp.minimum(sl[2], sl[3])),
            jnp.minimum(jnp.minimum(sl[4], sl[5]), jnp.minimum(sl[6], sl[7])))
        colid = jnp.full(v.shape, 7, jnp.int32)
        for j in range(6, -1, -1):
            colid = jnp.where(v == sl[j], j, colid)
        qb = jnp.bitwise_and(
            lax.bitcast_convert_type(v, jnp.int32) + 1024, ~_PM)
        fm = lax.bitcast_convert_type(
            jnp.bitwise_or(qb, fpos + jnp.left_shift(colid, 7)), jnp.float32)
        rk = lax.bitcast_convert_type(jnp.bitwise_or(jnp.bitwise_and(
            lax.bitcast_convert_type(run_k, jnp.int32), ~_PM), lane_k),
            jnp.float32)
        cand = jnp.concatenate([rk, fm], axis=1)
        new_k = init_k
        for t in range(_KV):
            mk = jnp.min(cand, axis=1, keepdims=True)
            new_k = jnp.where(lane_k == t, mk, new_k)
            if t + 1 < _KV:
                cand = jnp.where(cand == mk, _INF, cand)
        npos = jnp.bitwise_and(lax.bitcast_convert_type(new_k, jnp.int32), _PM)
        new_i = jnp.where(npos >= _K, c * _CB + (npos - _K), 0)
        for s in range(_KV):
            new_i = jnp.where(npos == s, run_i[:, s:s + 1], new_i)
        return new_k, new_i, S_next

    init = (init_k, jnp.zeros((_RB, _K), jnp.int32), _dot_chunk(0))
    run_k, run_i, _ = lax.fori_loop(0, nch, chunk_body, init)
    d2q = lax.bitcast_convert_type(
        jnp.bitwise_and(lax.bitcast_convert_type(run_k, jnp.int32), ~_PM),
        jnp.float32) - 1.0
    pdv = jnp.sqrt(jnp.maximum(d2q, 0.0))
    pd_ref[...] = jnp.where(lane_k < _KV, pdv, 0.0)
    idx_ref[...] = run_i


_topk_call = pl.pallas_call(
    _topk_body,
    grid=(_M // _RB,),
    in_specs=[
        pl.BlockSpec((_RB, _D), lambda i: (i, 0)),
        pl.BlockSpec((_D, _M), lambda i: (0, 0)),
    ],
    out_specs=[
        pl.BlockSpec((_RB, _K), lambda i: (i, 0)),
        pl.BlockSpec((_RB, _K), lambda i: (i, 0)),
    ],
    out_shape=[
        jax.ShapeDtypeStruct((_M, _K), jnp.float32),
        jax.ShapeDtypeStruct((_M, _K), jnp.int32),
    ],
    scratch_shapes=[pltpu.VMEM((1, _M), jnp.float32),
                    pltpu.VMEM((_D, _M), jnp.bfloat16),
                    pltpu.VMEM((_D, _M), jnp.bfloat16)],
)


def _sc_body(pd_hbm, idx_hbm, probs_hbm, lab_hbm, out_hbm,
             pd_v, idx_v, probs_v, lab_v, acc_v):
    nc = plsc.get_sparse_core_info().num_cores
    wid = lax.axis_index("s") * nc + lax.axis_index("c")
    pltpu.sync_copy(pd_hbm.at[wid], pd_v)
    pltpu.sync_copy(idx_hbm.at[wid], idx_v)
    pltpu.sync_copy(probs_hbm, probs_v)
    pltpu.sync_copy(lab_hbm, lab_v)
    base = wid * _RPW
    lanes = lax.iota(jnp.int32, 16)

    def group_body(g, acc):
        off = g * 16
        i_vec = base + off + lanes
        imod = jnp.bitwise_and(i_vec, _N - 1)
        lab_i = plsc.load_gather(lab_v, [imod])
        ok_i = lab_i != -1
        r = pd_v[pl.ds(_RPW + off, 16)]
        contrib = jnp.zeros((16,), jnp.float32)
        for k in range(_KV):
            pdk = pd_v[pl.ds(k * _RPW + off, 16)]
            idxk = idx_v[pl.ds(k * _RPW + off, 16)]
            jmod = jnp.bitwise_and(idxk, _N - 1)
            lab_j = plsc.load_gather(lab_v, [jmod])
            msk = (lab_i == lab_j) & (imod != jmod) & ok_i & (lab_j != -1)
            w = 1.0 - jnp.clip((pdk - r) / r, 0.0, 1.0)
            w = jnp.where(idxk == i_vec, 0.0, w)
            w = jnp.where(msk, w, 0.0)
            dot = jnp.zeros((16,), jnp.float32)
            for ch in range(_C):
                pic = plsc.load_gather(probs_v, [i_vec + ch * _M])
                pjc = plsc.load_gather(probs_v, [idxk + ch * _M])
                dot = dot + pic * pjc
            contrib = contrib + w * dot
        return acc + contrib

    acc = lax.fori_loop(0, _RPW // 16, group_body, jnp.zeros((16,), jnp.float32))
    acc_v[...] = acc
    pltpu.sync_copy(acc_v, out_hbm.at[wid])


def _make_sc_call():
    return functools.partial(
        pl.kernel,
        out_type=jax.ShapeDtypeStruct((_NW, 16), jnp.float32),
        mesh=plsc.VectorSubcoreMesh(core_axis_name="c", subcore_axis_name="s"),
        compiler_params=pltpu.CompilerParams(needs_layout_passes=False),
        scratch_types=[
            pltpu.VMEM((_K * _RPW,), jnp.float32),
            pltpu.VMEM((_K * _RPW,), jnp.int32),
            pltpu.VMEM((_C * _M,), jnp.float32),
            pltpu.VMEM((_N,), jnp.int32),
            pltpu.VMEM((16,), jnp.float32),
        ],
    )(_sc_body)


def kernel(z_i, z_j, z_i_prob, z_j_prob, pseudo_label):
    feats = jnp.concatenate([z_i, z_j], axis=0)
    probs = jnp.concatenate([z_i_prob, z_j_prob], axis=0)
    pd, idx = _topk_call(feats, feats.T)
    pd_w = pd.T.reshape(_K, _NW, _RPW).transpose(1, 0, 2).reshape(_NW, _K * _RPW)
    idx_w = idx.T.reshape(_K, _NW, _RPW).transpose(1, 0, 2).reshape(_NW, _K * _RPW)
    probs_f = probs.T.reshape(-1)
    partials = _make_sc_call()(pd_w, idx_w, probs_f, pseudo_label)
    return jnp.sum(partials) / jnp.float32(_M * _M)
